# Initial kernel scaffold; baseline (speedup 1.0000x reference)
import jax
import jax.numpy as jnp
from jax.experimental import pallas as pl


def kernel(x, edge_index, edge_attr, W_gcn, b_gcn, W_gat, att_src, att_dst, b_gat, W1a, b1a, W2a, b2a, W1v, b1v, W2v, b2v):
    raise NotImplementedError("write your pallas kernel here")


# speedup vs baseline: 15.7935x; 15.7935x over previous
import functools

import jax
import jax.numpy as jnp
from jax import lax
from jax.experimental import pallas as pl
from jax.experimental.pallas import tpu as pltpu
from jax.experimental.pallas import tpu_sc as plsc

_N = 10000
_D = 128
_NPAD = 10112
_NW = 32
_CH = 128
_NCH = 80
_EPW = _NCH * _CH
_EPAD = _NW * _EPW
_ZS = _NPAD // 16

_SLABS = ((0, 128), (128, 128), (256, 128), (384, 128), (512, 120))


def _zero_rows(rows, nrows):
    z16 = jnp.zeros((16,), jnp.float32)

    def z(r, carry):
        for k in range(_D // 16):
            rows[r, pl.ds(k * 16, 16)] = z16
        return carry

    lax.fori_loop(0, nrows, z, 0)


def _zero_acc2d(acc, rows, s):
    _zero_rows(rows, _CH)
    for off, sz in _SLABS:
        pltpu.sync_copy(rows.at[pl.ds(0, sz)], acc.at[pl.ds(s * _ZS + off, sz)])


def _drain_acc2d(acc, rows, s, out_ref):
    for off, sz in _SLABS:
        pltpu.sync_copy(acc.at[pl.ds(s * _ZS + off, sz)], rows.at[pl.ds(0, sz)])
        pltpu.sync_copy(rows.at[pl.ds(0, sz)], out_ref.at[pl.ds(s * _ZS + off, sz)])


def _zero_acc1d(acc, buf, s):
    z16 = jnp.zeros((16,), jnp.float32)

    def z(r, carry):
        buf[pl.ds(r * 16, 16)] = z16
        return carry

    lax.fori_loop(0, 640 // 16, z, 0)
    pltpu.sync_copy(buf.at[pl.ds(0, _ZS)], acc.at[pl.ds(s * _ZS, _ZS)])


def _drain_acc1d(acc, buf, s, c, out_hbm):
    pltpu.sync_copy(acc.at[pl.ds(s * _ZS, _ZS)], buf.at[pl.ds(0, _ZS)])
    pltpu.sync_copy(buf.at[pl.ds(0, _ZS)],
                    out_hbm.at[pl.ds(c * _NPAD + s * _ZS, _ZS)])


def _scale_rows(rows, wv):
    def group(i, carry):
        base = i * 16
        w16 = wv[pl.ds(base, 16)]
        for e in range(16):
            w = w16[e]
            for r in range(_D // 16):
                sl = pl.ds(r * 16, 16)
                rows[base + e, sl] = rows[base + e, sl] * w
        return carry

    lax.fori_loop(0, _CH // 16, group, 0)


_mesh = plsc.VectorSubcoreMesh(
    core_axis_name="c", subcore_axis_name="s", num_cores=2, num_subcores=16
)


@functools.partial(
    pl.kernel,
    out_type=jax.ShapeDtypeStruct((2 * _NPAD,), jnp.float32),
    mesh=_mesh,
    scratch_types=[
        pltpu.VMEM((_NCH, _CH), jnp.int32),
        pltpu.VMEM((_CH,), jnp.float32),
        pltpu.VMEM((640,), jnp.float32),
        pltpu.VMEM_SHARED((_NPAD,), jnp.float32),
    ],
)
def _deg_kernel(dst_hbm, out_hbm, dstv, onesv, zbuf, acc):
    c = lax.axis_index("c")
    s = lax.axis_index("s")
    wid = c * 16 + s
    _zero_acc1d(acc, zbuf, s)
    for i in range(_CH // 16):
        onesv[pl.ds(i * 16, 16)] = jnp.full((16,), 1.0, jnp.float32)
    pltpu.sync_copy(dst_hbm.at[wid], dstv)
    plsc.subcore_barrier()

    def chunk(j, carry):
        pltpu.sync_copy(onesv, acc.at[dstv.at[j]], add=True)
        return carry

    lax.fori_loop(0, _NCH, chunk, 0)
    plsc.subcore_barrier()
    _drain_acc1d(acc, zbuf, s, c, out_hbm)


@functools.partial(
    pl.kernel,
    out_type=jax.ShapeDtypeStruct((2, _NPAD, _D), jnp.float32),
    mesh=_mesh,
    scratch_types=[
        pltpu.VMEM((_NCH, _CH), jnp.int32),
        pltpu.VMEM((_NCH, _CH), jnp.int32),
        pltpu.VMEM((_CH,), jnp.float32),
        pltpu.VMEM((_CH, _D), jnp.float32),
        pltpu.SemaphoreType.DMA,
        pltpu.SemaphoreType.DMA,
        pltpu.VMEM_SHARED((_NPAD, _D), jnp.float32),
    ],
)
def _gcn_kernel(h_hbm, dis_hbm, src_hbm, dst_hbm, out_hbm,
                srcv, dstv, wv, rows, sem1, sem2, acc):
    c = lax.axis_index("c")
    s = lax.axis_index("s")
    wid = c * 16 + s
    _zero_acc2d(acc, rows, s)
    pltpu.sync_copy(src_hbm.at[wid], srcv)
    pltpu.sync_copy(dst_hbm.at[wid], dstv)
    plsc.subcore_barrier()

    def chunk(j, carry):
        idx = srcv.at[j]
        cp1 = pltpu.async_copy(h_hbm.at[idx], rows, sem1)
        cp2 = pltpu.async_copy(dis_hbm.at[idx], wv, sem2)
        cp1.wait()
        cp2.wait()
        _scale_rows(rows, wv)
        pltpu.sync_copy(rows, acc.at[dstv.at[j]], add=True)
        return carry

    lax.fori_loop(0, _NCH, chunk, 0)
    plsc.subcore_barrier()
    _drain_acc2d(acc, rows, s, out_hbm.at[c])


@functools.partial(
    pl.kernel,
    out_type=(
        jax.ShapeDtypeStruct((2 * _NPAD,), jnp.float32),
        jax.ShapeDtypeStruct((2, _NPAD, _D), jnp.float32),
    ),
    mesh=_mesh,
    scratch_types=[
        pltpu.VMEM((_NCH, _CH), jnp.int32),
        pltpu.VMEM((_NCH, _CH), jnp.int32),
        pltpu.VMEM((_CH,), jnp.float32),
        pltpu.VMEM((_CH,), jnp.float32),
        pltpu.VMEM((_CH,), jnp.float32),
        pltpu.VMEM((_CH, _D), jnp.float32),
        pltpu.VMEM((16,), jnp.float32),
        pltpu.VMEM((640,), jnp.float32),
        pltpu.SemaphoreType.DMA,
        pltpu.SemaphoreType.DMA,
        pltpu.SemaphoreType.DMA,
        pltpu.VMEM_SHARED((_NPAD,), jnp.float32),
        pltpu.VMEM_SHARED((_NPAD, _D), jnp.float32),
    ],
)
def _gat_kernel(h2_hbm, asrc_hbm, adst_hbm, mb_hbm, src_hbm, dst_hbm,
                outS_hbm, outN_hbm,
                srcv, dstv, asv, adv, pv, rows, mv, zbuf, sem1, sem2, sem3,
                accS, accN):
    c = lax.axis_index("c")
    s = lax.axis_index("s")
    wid = c * 16 + s
    _zero_acc1d(accS, zbuf, s)
    _zero_acc2d(accN, rows, s)
    pltpu.sync_copy(mb_hbm, mv)
    pltpu.sync_copy(src_hbm.at[wid], srcv)
    pltpu.sync_copy(dst_hbm.at[wid], dstv)
    plsc.subcore_barrier()

    def chunk(j, carry):
        sidx = srcv.at[j]
        didx = dstv.at[j]
        cp1 = pltpu.async_copy(h2_hbm.at[sidx], rows, sem1)
        cp2 = pltpu.async_copy(asrc_hbm.at[sidx], asv, sem2)
        cp3 = pltpu.async_copy(adst_hbm.at[didx], adv, sem3)
        cp1.wait()
        cp2.wait()
        cp3.wait()
        m16 = mv[...]
        for i in range(_CH // 16):
            sl = pl.ds(i * 16, 16)
            sc = asv[sl] + adv[sl]
            sc = jnp.where(sc >= 0.0, sc, 0.2 * sc)
            pv[sl] = jnp.exp(sc - m16)
        _scale_rows(rows, pv)
        pltpu.sync_copy(pv, accS.at[didx], add=True)
        pltpu.sync_copy(rows, accN.at[didx], add=True)
        return carry

    lax.fori_loop(0, _NCH, chunk, 0)
    plsc.subcore_barrier()
    _drain_acc1d(accS, zbuf, s, c, outS_hbm)
    _drain_acc2d(accN, rows, s, outN_hbm.at[c])


def _t1_body(x_ref, w_ref, degp_ref, h_ref, dis_ref):
    h_ref[...] = jnp.dot(x_ref[...], w_ref[...], preferred_element_type=jnp.float32)
    deg = degp_ref[0] + degp_ref[1] + 1.0
    dis_ref[...] = lax.rsqrt(deg)


def _t2_body(h_ref, dis_ref, a0_ref, a1_ref, bg_ref, wgat_ref, vs_ref, vd_ref,
             h2_ref, asrc_ref, adst_ref, mb_ref, pself_ref):
    dis = dis_ref[...]
    h = h_ref[...]
    agg = dis[:, None] * (a0_ref[...] + a1_ref[...]) + (dis * dis)[:, None] * h
    h1 = jnp.maximum(agg + bg_ref[...][None, :], 0.0)
    h2 = jnp.dot(h1, wgat_ref[...], preferred_element_type=jnp.float32)
    h2_ref[...] = h2
    asrc = jnp.dot(h2, vs_ref[...][:, None], preferred_element_type=jnp.float32)[:, 0]
    adst = jnp.dot(h2, vd_ref[...][:, None], preferred_element_type=jnp.float32)[:, 0]
    asrc_ref[...] = asrc
    adst_ref[...] = adst
    mraw = jnp.max(asrc) + jnp.max(adst)
    m = jnp.where(mraw >= 0.0, mraw, 0.2 * mraw)
    mb_ref[...] = jnp.full((16,), m, jnp.float32)
    sself = asrc + adst
    sself = jnp.where(sself >= 0.0, sself, 0.2 * sself)
    pself_ref[...] = jnp.exp(sself - m)


def _t3_body(sp_ref, np_ref, h2_ref, pself_ref, bgat_ref,
             w1a_ref, b1a_ref, w2a_ref, b2a_ref,
             w1v_ref, b1v_ref, w2v_ref, b2v_ref, q_ref):
    pself = pself_ref[...]
    ssum = sp_ref[0] + sp_ref[1] + pself
    numer = np_ref[0] + np_ref[1] + pself[:, None] * h2_ref[...]
    h2 = jnp.maximum(numer / (ssum[:, None] + 1e-16) + bgat_ref[...][None, :], 0.0)
    g = jnp.sum(h2[:_N], axis=0, keepdims=True)
    ga = jnp.maximum(jnp.dot(g, w1a_ref[...], preferred_element_type=jnp.float32)
                     + b1a_ref[...][None, :], 0.0)
    a = jnp.dot(ga, w2a_ref[...], preferred_element_type=jnp.float32) + b2a_ref[...][None, :]
    gv = jnp.maximum(jnp.dot(g, w1v_ref[...], preferred_element_type=jnp.float32)
                     + b1v_ref[...][None, :], 0.0)
    v = jnp.dot(gv, w2v_ref[...], preferred_element_type=jnp.float32) + b2v_ref[...][None, :]
    q_ref[...] = v + a - jnp.mean(a, axis=1, keepdims=True)


def kernel(x, edge_index, edge_attr, W_gcn, b_gcn, W_gat, att_src, att_dst,
           b_gat, W1a, b1a, W2a, b2a, W1v, b1v, W2v, b2v):
    del edge_attr
    e = edge_index.shape[1]
    pad_e = _EPAD - e
    src = jnp.concatenate([edge_index[0], jnp.full((pad_e,), _N, jnp.int32)])
    dst = jnp.concatenate([edge_index[1], jnp.full((pad_e,), _N, jnp.int32)])
    src_r = src.reshape(_NW, _NCH, _CH)
    dst_r = dst.reshape(_NW, _NCH, _CH)
    xp = jnp.concatenate([x, jnp.zeros((_NPAD - _N, _D), jnp.float32)], axis=0)

    degp = _deg_kernel(dst_r).reshape(2, _NPAD)

    h_gcn, dis = pl.pallas_call(
        _t1_body,
        out_shape=(
            jax.ShapeDtypeStruct((_NPAD, _D), jnp.float32),
            jax.ShapeDtypeStruct((_NPAD,), jnp.float32),
        ),
    )(xp, W_gcn, degp)

    accg = _gcn_kernel(h_gcn, dis, src_r, dst_r)

    h2pre, asrc, adst, mb, pself = pl.pallas_call(
        _t2_body,
        out_shape=(
            jax.ShapeDtypeStruct((_NPAD, _D), jnp.float32),
            jax.ShapeDtypeStruct((_NPAD,), jnp.float32),
            jax.ShapeDtypeStruct((_NPAD,), jnp.float32),
            jax.ShapeDtypeStruct((16,), jnp.float32),
            jax.ShapeDtypeStruct((_NPAD,), jnp.float32),
        ),
    )(h_gcn, dis, accg[0], accg[1], b_gcn, W_gat, att_src, att_dst)

    ssump, numerp = _gat_kernel(h2pre, asrc, adst, mb, src_r, dst_r)
    ssump = ssump.reshape(2, _NPAD)

    q = pl.pallas_call(
        _t3_body,
        out_shape=jax.ShapeDtypeStruct((1, 5), jnp.float32),
    )(ssump, numerp, h2pre, pself, b_gat,
      W1a, b1a, W2a, b2a, W1v, b1v, W2v, b2v)
    return q


# speedup vs baseline: 19.0097x; 1.2036x over previous
import functools

import jax
import jax.numpy as jnp
from jax import lax
from jax.experimental import pallas as pl
from jax.experimental.pallas import tpu as pltpu
from jax.experimental.pallas import tpu_sc as plsc

_N = 10000
_D = 128
_NPAD = 10112
_NW = 32
_CH = 128
_NCH = 80
_EPW = _NCH * _CH
_EPAD = _NW * _EPW
_ZS = _NPAD // 16

_SLABS = tuple((off, min(_CH, _ZS - off)) for off in range(0, _ZS, _CH))


def _zero_rows(rows, nrows):
    z16 = jnp.zeros((16,), jnp.float32)

    def z(r, carry):
        for k in range(_D // 16):
            rows[r, pl.ds(k * 16, 16)] = z16
        return carry

    lax.fori_loop(0, nrows, z, 0)


def _zero_acc2d(acc, rows, s):
    _zero_rows(rows, _CH)
    for off, sz in _SLABS:
        pltpu.sync_copy(rows.at[pl.ds(0, sz)], acc.at[pl.ds(s * _ZS + off, sz)])


def _drain_acc2d(acc, rows, s, out_ref):
    for off, sz in _SLABS:
        pltpu.sync_copy(acc.at[pl.ds(s * _ZS + off, sz)], rows.at[pl.ds(0, sz)])
        pltpu.sync_copy(rows.at[pl.ds(0, sz)], out_ref.at[pl.ds(s * _ZS + off, sz)])


def _zero_acc1d(acc, buf, s):
    z16 = jnp.zeros((16,), jnp.float32)

    def z(r, carry):
        buf[pl.ds(r * 16, 16)] = z16
        return carry

    lax.fori_loop(0, 640 // 16, z, 0)
    pltpu.sync_copy(buf.at[pl.ds(0, _ZS)], acc.at[pl.ds(s * _ZS, _ZS)])


def _drain_acc1d(acc, buf, s, c, out_hbm):
    pltpu.sync_copy(acc.at[pl.ds(s * _ZS, _ZS)], buf.at[pl.ds(0, _ZS)])
    pltpu.sync_copy(buf.at[pl.ds(0, _ZS)],
                    out_hbm.at[pl.ds(c * _NPAD + s * _ZS, _ZS)])


def _scale_rows(rows, wv):
    def group(i, carry):
        base = i * 16
        w16 = wv[pl.ds(base, 16)]
        for e in range(16):
            w = w16[e]
            for r in range(_D // 16):
                sl = pl.ds(r * 16, 16)
                rows[base + e, sl] = rows[base + e, sl] * w
        return carry

    lax.fori_loop(0, _CH // 16, group, 0)


_mesh = plsc.VectorSubcoreMesh(
    core_axis_name="c", subcore_axis_name="s", num_cores=2, num_subcores=16
)


@functools.partial(
    pl.kernel,
    out_type=jax.ShapeDtypeStruct((2 * _NPAD,), jnp.float32),
    mesh=_mesh,
    scratch_types=[
        pltpu.VMEM((_NCH, _CH), jnp.int32),
        pltpu.VMEM((_CH,), jnp.float32),
        pltpu.VMEM((640,), jnp.float32),
        pltpu.VMEM_SHARED((_NPAD,), jnp.float32),
    ],
)
def _deg_kernel(dst_hbm, out_hbm, dstv, onesv, zbuf, acc):
    c = lax.axis_index("c")
    s = lax.axis_index("s")
    wid = c * 16 + s
    _zero_acc1d(acc, zbuf, s)
    for i in range(_CH // 16):
        onesv[pl.ds(i * 16, 16)] = jnp.full((16,), 1.0, jnp.float32)
    pltpu.sync_copy(dst_hbm.at[wid], dstv)
    plsc.subcore_barrier()

    def chunk(j, carry):
        pltpu.sync_copy(onesv, acc.at[dstv.at[j]], add=True)
        return carry

    lax.fori_loop(0, _NCH, chunk, 0)
    plsc.subcore_barrier()
    _drain_acc1d(acc, zbuf, s, c, out_hbm)


@functools.partial(
    pl.kernel,
    out_type=jax.ShapeDtypeStruct((2, _NPAD, _D), jnp.float32),
    mesh=_mesh,
    scratch_types=[
        pltpu.VMEM((_NCH, _CH), jnp.int32),
        pltpu.VMEM((2, _CH), jnp.int32),
        pltpu.VMEM((_CH, _D), jnp.float32),
        pltpu.VMEM((_CH, _D), jnp.float32),
        pltpu.SemaphoreType.DMA,
        pltpu.SemaphoreType.DMA,
        pltpu.SemaphoreType.DMA,
        pltpu.SemaphoreType.DMA,
        pltpu.VMEM_SHARED((_NPAD, _D), jnp.float32),
    ],
)
def _gcn_kernel(hs_hbm, srcf_hbm, dst_hbm, out_hbm,
                dstv, srcb, rows0, rows1, semIA, semIB, semRA, semRB, acc):
    c = lax.axis_index("c")
    s = lax.axis_index("s")
    wid = c * 16 + s
    _zero_acc2d(acc, rows0, s)
    pltpu.sync_copy(dst_hbm.at[wid], dstv)
    base = wid * _NCH * _CH

    def idx_src(j):
        return srcf_hbm.at[pl.ds(base + j * _CH, _CH)]

    pltpu.async_copy(idx_src(0), srcb.at[0], semIA)
    pltpu.async_copy(idx_src(1), srcb.at[1], semIB)
    plsc.subcore_barrier()
    pltpu.make_async_copy(idx_src(0), srcb.at[0], semIA).wait()
    pltpu.async_copy(hs_hbm.at[srcb.at[0]], rows0, semRA)

    def pair(t, carry):
        j0 = 2 * t
        j1 = j0 + 1
        pltpu.make_async_copy(idx_src(j1), srcb.at[1], semIB).wait()
        pltpu.async_copy(hs_hbm.at[srcb.at[1]], rows1, semRB)
        pltpu.make_async_copy(hs_hbm.at[srcb.at[0]], rows0, semRA).wait()

        @pl.when(t < _NCH // 2 - 1)
        def _():
            pltpu.async_copy(idx_src(j0 + 2), srcb.at[0], semIA)

        pltpu.sync_copy(rows0, acc.at[dstv.at[j0]], add=True)

        @pl.when(t < _NCH // 2 - 1)
        def _():
            pltpu.make_async_copy(idx_src(j0 + 2), srcb.at[0], semIA).wait()
            pltpu.async_copy(hs_hbm.at[srcb.at[0]], rows0, semRA)

        pltpu.make_async_copy(hs_hbm.at[srcb.at[1]], rows1, semRB).wait()

        @pl.when(t < _NCH // 2 - 1)
        def _():
            pltpu.async_copy(idx_src(j1 + 2), srcb.at[1], semIB)

        pltpu.sync_copy(rows1, acc.at[dstv.at[j1]], add=True)
        return carry

    lax.fori_loop(0, _NCH // 2, pair, 0)
    plsc.subcore_barrier()
    _drain_acc2d(acc, rows0, s, out_hbm.at[c])


@functools.partial(
    pl.kernel,
    out_type=(
        jax.ShapeDtypeStruct((2 * _NPAD,), jnp.float32),
        jax.ShapeDtypeStruct((2, _NPAD, _D), jnp.float32),
    ),
    mesh=_mesh,
    scratch_types=[
        pltpu.VMEM((_NCH, _CH), jnp.int32),
        pltpu.VMEM((2, _CH), jnp.int32),
        pltpu.VMEM((_CH,), jnp.float32),
        pltpu.VMEM((_CH,), jnp.float32),
        pltpu.VMEM((_CH,), jnp.float32),
        pltpu.VMEM((_CH,), jnp.float32),
        pltpu.VMEM((_CH,), jnp.float32),
        pltpu.VMEM((_CH, _D), jnp.float32),
        pltpu.VMEM((_CH, _D), jnp.float32),
        pltpu.VMEM((16,), jnp.float32),
        pltpu.VMEM((640,), jnp.float32),
        pltpu.SemaphoreType.DMA,
        pltpu.SemaphoreType.DMA,
        pltpu.SemaphoreType.DMA,
        pltpu.SemaphoreType.DMA,
        pltpu.VMEM_SHARED((_NPAD,), jnp.float32),
        pltpu.VMEM_SHARED((_NPAD, _D), jnp.float32),
    ],
)
def _gat_kernel(h2_hbm, asrc_hbm, adst_hbm, mb_hbm, srcf_hbm, dst_hbm,
                outS_hbm, outN_hbm,
                dstv, srcb, asv0, adv0, asv1, adv1, pv, rows0, rows1, mv,
                zbuf, semIA, semIB, semRA, semRB, accS, accN):
    c = lax.axis_index("c")
    s = lax.axis_index("s")
    wid = c * 16 + s
    _zero_acc1d(accS, zbuf, s)
    _zero_acc2d(accN, rows0, s)
    pltpu.sync_copy(mb_hbm, mv)
    pltpu.sync_copy(dst_hbm.at[wid], dstv)
    base = wid * _NCH * _CH

    def idx_src(j):
        return srcf_hbm.at[pl.ds(base + j * _CH, _CH)]

    def fire(j, b, rows_b, asv_b, adv_b, sem):
        pltpu.async_copy(h2_hbm.at[srcb.at[b]], rows_b, sem)
        pltpu.async_copy(asrc_hbm.at[srcb.at[b]], asv_b, sem)
        pltpu.async_copy(adst_hbm.at[dstv.at[j]], adv_b, sem)

    def drain(j, b, rows_b, asv_b, adv_b, sem):
        pltpu.make_async_copy(h2_hbm.at[srcb.at[b]], rows_b, sem).wait()
        pltpu.make_async_copy(asrc_hbm.at[srcb.at[b]], asv_b, sem).wait()
        pltpu.make_async_copy(adst_hbm.at[dstv.at[j]], adv_b, sem).wait()

    def compute(j, rows_b, asv_b, adv_b):
        m16 = mv[...]
        for i in range(_CH // 16):
            sl = pl.ds(i * 16, 16)
            sc = asv_b[sl] + adv_b[sl]
            sc = jnp.where(sc >= 0.0, sc, 0.2 * sc)
            pv[sl] = jnp.exp(sc - m16)
        _scale_rows(rows_b, pv)
        pltpu.sync_copy(pv, accS.at[dstv.at[j]], add=True)
        pltpu.sync_copy(rows_b, accN.at[dstv.at[j]], add=True)

    pltpu.async_copy(idx_src(0), srcb.at[0], semIA)
    pltpu.async_copy(idx_src(1), srcb.at[1], semIB)
    plsc.subcore_barrier()
    pltpu.make_async_copy(idx_src(0), srcb.at[0], semIA).wait()
    fire(0, 0, rows0, asv0, adv0, semRA)

    def pair(t, carry):
        j0 = 2 * t
        j1 = j0 + 1
        pltpu.make_async_copy(idx_src(j1), srcb.at[1], semIB).wait()
        fire(j1, 1, rows1, asv1, adv1, semRB)
        drain(j0, 0, rows0, asv0, adv0, semRA)

        @pl.when(t < _NCH // 2 - 1)
        def _():
            pltpu.async_copy(idx_src(j0 + 2), srcb.at[0], semIA)

        compute(j0, rows0, asv0, adv0)

        @pl.when(t < _NCH // 2 - 1)
        def _():
            pltpu.make_async_copy(idx_src(j0 + 2), srcb.at[0], semIA).wait()
            fire(j0 + 2, 0, rows0, asv0, adv0, semRA)

        drain(j1, 1, rows1, asv1, adv1, semRB)

        @pl.when(t < _NCH // 2 - 1)
        def _():
            pltpu.async_copy(idx_src(j1 + 2), srcb.at[1], semIB)

        compute(j1, rows1, asv1, adv1)
        return carry

    lax.fori_loop(0, _NCH // 2, pair, 0)
    plsc.subcore_barrier()
    _drain_acc1d(accS, zbuf, s, c, outS_hbm)
    _drain_acc2d(accN, rows0, s, outN_hbm.at[c])


def _t1_body(x_ref, w_ref, degp_ref, hs_ref, dis_ref):
    h = jnp.dot(x_ref[...], w_ref[...], preferred_element_type=jnp.float32)
    deg = degp_ref[0] + degp_ref[1] + 1.0
    dis = lax.rsqrt(deg)
    dis_ref[...] = dis
    hs_ref[...] = dis[:, None] * h


def _t2_body(hs_ref, dis_ref, a0_ref, a1_ref, bg_ref, wgat_ref, vs_ref, vd_ref,
             h2_ref, asrc_ref, adst_ref, mb_ref, pself_ref):
    dis = dis_ref[...]
    agg = dis[:, None] * (a0_ref[...] + a1_ref[...] + hs_ref[...])
    h1 = jnp.maximum(agg + bg_ref[...][None, :], 0.0)
    h2 = jnp.dot(h1, wgat_ref[...], preferred_element_type=jnp.float32)
    h2_ref[...] = h2
    asrc = jnp.dot(h2, vs_ref[...][:, None], preferred_element_type=jnp.float32)[:, 0]
    adst = jnp.dot(h2, vd_ref[...][:, None], preferred_element_type=jnp.float32)[:, 0]
    asrc_ref[...] = asrc
    adst_ref[...] = adst
    mraw = jnp.max(asrc) + jnp.max(adst)
    m = jnp.where(mraw >= 0.0, mraw, 0.2 * mraw)
    mb_ref[...] = jnp.full((16,), m, jnp.float32)
    sself = asrc + adst
    sself = jnp.where(sself >= 0.0, sself, 0.2 * sself)
    pself_ref[...] = jnp.exp(sself - m)


def _t3_body(sp_ref, np_ref, h2_ref, pself_ref, bgat_ref,
             w1a_ref, b1a_ref, w2a_ref, b2a_ref,
             w1v_ref, b1v_ref, w2v_ref, b2v_ref, q_ref):
    pself = pself_ref[...]
    ssum = sp_ref[0] + sp_ref[1] + pself
    numer = np_ref[0] + np_ref[1] + pself[:, None] * h2_ref[...]
    h2 = jnp.maximum(numer / (ssum[:, None] + 1e-16) + bgat_ref[...][None, :], 0.0)
    g = jnp.sum(h2[:_N], axis=0, keepdims=True)
    ga = jnp.maximum(jnp.dot(g, w1a_ref[...], preferred_element_type=jnp.float32)
                     + b1a_ref[...][None, :], 0.0)
    a = jnp.dot(ga, w2a_ref[...], preferred_element_type=jnp.float32) + b2a_ref[...][None, :]
    gv = jnp.maximum(jnp.dot(g, w1v_ref[...], preferred_element_type=jnp.float32)
                     + b1v_ref[...][None, :], 0.0)
    v = jnp.dot(gv, w2v_ref[...], preferred_element_type=jnp.float32) + b2v_ref[...][None, :]
    q_ref[...] = v + a - jnp.mean(a, axis=1, keepdims=True)


def kernel(x, edge_index, edge_attr, W_gcn, b_gcn, W_gat, att_src, att_dst,
           b_gat, W1a, b1a, W2a, b2a, W1v, b1v, W2v, b2v):
    del edge_attr
    e = edge_index.shape[1]
    pad_e = _EPAD - e
    src = jnp.concatenate([edge_index[0], jnp.full((pad_e,), _N, jnp.int32)])
    dst = jnp.concatenate([edge_index[1], jnp.full((pad_e,), _N, jnp.int32)])
    dst_r = dst.reshape(_NW, _NCH, _CH)
    xp = jnp.concatenate([x, jnp.zeros((_NPAD - _N, _D), jnp.float32)], axis=0)

    degp = _deg_kernel(dst_r).reshape(2, _NPAD)

    hs, dis = pl.pallas_call(
        _t1_body,
        out_shape=(
            jax.ShapeDtypeStruct((_NPAD, _D), jnp.float32),
            jax.ShapeDtypeStruct((_NPAD,), jnp.float32),
        ),
    )(xp, W_gcn, degp)

    accg = _gcn_kernel(hs, src, dst_r)

    h2pre, asrc, adst, mb, pself = pl.pallas_call(
        _t2_body,
        out_shape=(
            jax.ShapeDtypeStruct((_NPAD, _D), jnp.float32),
            jax.ShapeDtypeStruct((_NPAD,), jnp.float32),
            jax.ShapeDtypeStruct((_NPAD,), jnp.float32),
            jax.ShapeDtypeStruct((16,), jnp.float32),
            jax.ShapeDtypeStruct((_NPAD,), jnp.float32),
        ),
    )(hs, dis, accg[0], accg[1], b_gcn, W_gat, att_src, att_dst)

    ssump, numerp = _gat_kernel(h2pre, asrc, adst, mb, src, dst_r)
    ssump = ssump.reshape(2, _NPAD)

    q = pl.pallas_call(
        _t3_body,
        out_shape=jax.ShapeDtypeStruct((1, 5), jnp.float32),
    )(ssump, numerp, h2pre, pself, b_gat,
      W1a, b1a, W2a, b2a, W1v, b1v, W2v, b2v)
    return q


# speedup vs baseline: 55.7770x; 2.9341x over previous
import functools

import jax
import jax.numpy as jnp
from jax import lax
from jax.experimental import pallas as pl
from jax.experimental.pallas import tpu as pltpu
from jax.experimental.pallas import tpu_sc as plsc

_N = 10000
_D = 128
_NPAD = 10112
_NW = 32
_CH = 128
_NCH = 80
_EPW = _NCH * _CH
_EPAD = _NW * _EPW
_ZS = _NPAD // 16

_SLABS = tuple((off, min(_CH, _ZS - off)) for off in range(0, _ZS, _CH))


def _zero_rows(rows, nrows):
    z16 = jnp.zeros((16,), jnp.float32)

    def z(r, carry):
        for k in range(_D // 16):
            rows[r, pl.ds(k * 16, 16)] = z16
        return carry

    lax.fori_loop(0, nrows, z, 0)


def _zero_acc2d(acc, rows, s):
    _zero_rows(rows, _CH)
    for off, sz in _SLABS:
        pltpu.sync_copy(rows.at[pl.ds(0, sz)], acc.at[pl.ds(s * _ZS + off, sz)])


def _drain_acc2d(acc, rows, s, out_ref):
    for off, sz in _SLABS:
        pltpu.sync_copy(acc.at[pl.ds(s * _ZS + off, sz)], rows.at[pl.ds(0, sz)])
        pltpu.sync_copy(rows.at[pl.ds(0, sz)], out_ref.at[pl.ds(s * _ZS + off, sz)])


def _zero_acc1d(acc, buf, s):
    z16 = jnp.zeros((16,), jnp.float32)

    def z(r, carry):
        buf[pl.ds(r * 16, 16)] = z16
        return carry

    lax.fori_loop(0, 640 // 16, z, 0)
    pltpu.sync_copy(buf.at[pl.ds(0, _ZS)], acc.at[pl.ds(s * _ZS, _ZS)])


def _drain_acc1d(acc, buf, s, c, out_hbm):
    pltpu.sync_copy(acc.at[pl.ds(s * _ZS, _ZS)], buf.at[pl.ds(0, _ZS)])
    pltpu.sync_copy(buf.at[pl.ds(0, _ZS)],
                    out_hbm.at[pl.ds(c * _NPAD + s * _ZS, _ZS)])


def _scale_rows(rows, wv):
    def group(i, carry):
        base = i * 16
        w16 = wv[pl.ds(base, 16)]
        for e in range(16):
            w = w16[e]
            for r in range(_D // 16):
                sl = pl.ds(r * 16, 16)
                rows[base + e, sl] = rows[base + e, sl] * w
        return carry

    lax.fori_loop(0, _CH // 16, group, 0)


_mesh = plsc.VectorSubcoreMesh(
    core_axis_name="c", subcore_axis_name="s", num_cores=2, num_subcores=16
)


@functools.partial(
    pl.kernel,
    out_type=jax.ShapeDtypeStruct((2 * _NPAD,), jnp.float32),
    mesh=_mesh,
    scratch_types=[
        pltpu.VMEM((_NCH, _CH), jnp.int32),
        pltpu.VMEM((_CH,), jnp.float32),
        pltpu.VMEM((640,), jnp.float32),
        pltpu.VMEM_SHARED((_NPAD,), jnp.float32),
    ],
)
def _deg_kernel(dst_hbm, out_hbm, dstv, onesv, zbuf, acc):
    c = lax.axis_index("c")
    s = lax.axis_index("s")
    wid = c * 16 + s
    _zero_acc1d(acc, zbuf, s)
    for i in range(_CH // 16):
        onesv[pl.ds(i * 16, 16)] = jnp.full((16,), 1.0, jnp.float32)
    pltpu.sync_copy(dst_hbm.at[wid], dstv)
    plsc.subcore_barrier()

    def chunk(j, carry):
        pltpu.sync_copy(onesv, acc.at[dstv.at[j]], add=True)
        return carry

    lax.fori_loop(0, _NCH, chunk, 0)
    plsc.subcore_barrier()
    _drain_acc1d(acc, zbuf, s, c, out_hbm)


@functools.partial(
    pl.kernel,
    out_type=jax.ShapeDtypeStruct((2, _NPAD, _D), jnp.float32),
    mesh=_mesh,
    scratch_types=[
        pltpu.VMEM((_NCH, _CH), jnp.int32),
        pltpu.VMEM((2, _CH), jnp.int32),
        pltpu.VMEM((_CH, _D), jnp.float32),
        pltpu.VMEM((_CH, _D), jnp.float32),
        pltpu.SemaphoreType.DMA,
        pltpu.SemaphoreType.DMA,
        pltpu.SemaphoreType.DMA,
        pltpu.SemaphoreType.DMA,
        pltpu.VMEM_SHARED((_NPAD, _D), jnp.float32),
    ],
)
def _gcn_kernel(hs_hbm, srcf_hbm, dst_hbm, out_hbm,
                dstv, srcb, rows0, rows1, semIA, semIB, semRA, semRB, acc):
    c = lax.axis_index("c")
    s = lax.axis_index("s")
    wid = c * 16 + s
    _zero_acc2d(acc, rows0, s)
    pltpu.sync_copy(dst_hbm.at[wid], dstv)
    base = wid * _NCH * _CH

    def idx_src(j):
        return srcf_hbm.at[pl.ds(base + j * _CH, _CH)]

    pltpu.async_copy(idx_src(0), srcb.at[0], semIA)
    pltpu.async_copy(idx_src(1), srcb.at[1], semIB)
    plsc.subcore_barrier()
    pltpu.make_async_copy(idx_src(0), srcb.at[0], semIA).wait()
    pltpu.async_copy(hs_hbm.at[srcb.at[0]], rows0, semRA)

    def pair(t, carry):
        j0 = 2 * t
        j1 = j0 + 1
        pltpu.make_async_copy(idx_src(j1), srcb.at[1], semIB).wait()
        pltpu.async_copy(hs_hbm.at[srcb.at[1]], rows1, semRB)
        pltpu.make_async_copy(hs_hbm.at[srcb.at[0]], rows0, semRA).wait()

        @pl.when(t < _NCH // 2 - 1)
        def _():
            pltpu.async_copy(idx_src(j0 + 2), srcb.at[0], semIA)

        pltpu.sync_copy(rows0, acc.at[dstv.at[j0]], add=True)

        @pl.when(t < _NCH // 2 - 1)
        def _():
            pltpu.make_async_copy(idx_src(j0 + 2), srcb.at[0], semIA).wait()
            pltpu.async_copy(hs_hbm.at[srcb.at[0]], rows0, semRA)

        pltpu.make_async_copy(hs_hbm.at[srcb.at[1]], rows1, semRB).wait()

        @pl.when(t < _NCH // 2 - 1)
        def _():
            pltpu.async_copy(idx_src(j1 + 2), srcb.at[1], semIB)

        pltpu.sync_copy(rows1, acc.at[dstv.at[j1]], add=True)
        return carry

    lax.fori_loop(0, _NCH // 2, pair, 0)
    plsc.subcore_barrier()
    _drain_acc2d(acc, rows0, s, out_hbm.at[c])


@functools.partial(
    pl.kernel,
    out_type=(
        jax.ShapeDtypeStruct((2 * _NPAD,), jnp.float32),
        jax.ShapeDtypeStruct((2, _NPAD, _D), jnp.float32),
    ),
    mesh=_mesh,
    scratch_types=[
        pltpu.VMEM((_NCH, _CH), jnp.int32),
        pltpu.VMEM((2, _CH), jnp.int32),
        pltpu.VMEM((_CH,), jnp.float32),
        pltpu.VMEM((_CH,), jnp.float32),
        pltpu.VMEM((_CH,), jnp.float32),
        pltpu.VMEM((_CH,), jnp.float32),
        pltpu.VMEM((_CH,), jnp.float32),
        pltpu.VMEM((_CH, _D), jnp.float32),
        pltpu.VMEM((_CH, _D), jnp.float32),
        pltpu.VMEM((16,), jnp.float32),
        pltpu.VMEM((640,), jnp.float32),
        pltpu.SemaphoreType.DMA,
        pltpu.SemaphoreType.DMA,
        pltpu.SemaphoreType.DMA,
        pltpu.SemaphoreType.DMA,
        pltpu.VMEM_SHARED((_NPAD,), jnp.float32),
        pltpu.VMEM_SHARED((_NPAD, _D), jnp.float32),
    ],
)
def _gat_kernel(h2_hbm, asrc_hbm, adst_hbm, mb_hbm, srcf_hbm, dst_hbm,
                outS_hbm, outN_hbm,
                dstv, srcb, asv0, adv0, asv1, adv1, pv, rows0, rows1, mv,
                zbuf, semIA, semIB, semRA, semRB, accS, accN):
    c = lax.axis_index("c")
    s = lax.axis_index("s")
    wid = c * 16 + s
    _zero_acc1d(accS, zbuf, s)
    _zero_acc2d(accN, rows0, s)
    pltpu.sync_copy(mb_hbm, mv)
    pltpu.sync_copy(dst_hbm.at[wid], dstv)
    base = wid * _NCH * _CH

    def idx_src(j):
        return srcf_hbm.at[pl.ds(base + j * _CH, _CH)]

    def fire(j, b, rows_b, asv_b, adv_b, sem):
        pltpu.async_copy(h2_hbm.at[srcb.at[b]], rows_b, sem)
        pltpu.async_copy(asrc_hbm.at[srcb.at[b]], asv_b, sem)
        pltpu.async_copy(adst_hbm.at[dstv.at[j]], adv_b, sem)

    def drain(j, b, rows_b, asv_b, adv_b, sem):
        pltpu.make_async_copy(h2_hbm.at[srcb.at[b]], rows_b, sem).wait()
        pltpu.make_async_copy(asrc_hbm.at[srcb.at[b]], asv_b, sem).wait()
        pltpu.make_async_copy(adst_hbm.at[dstv.at[j]], adv_b, sem).wait()

    def compute(j, rows_b, asv_b, adv_b):
        m16 = mv[...]
        for i in range(_CH // 16):
            sl = pl.ds(i * 16, 16)
            sc = asv_b[sl] + adv_b[sl]
            sc = jnp.where(sc >= 0.0, sc, 0.2 * sc)
            pv[sl] = jnp.exp(sc - m16)
        _scale_rows(rows_b, pv)
        pltpu.sync_copy(pv, accS.at[dstv.at[j]], add=True)
        pltpu.sync_copy(rows_b, accN.at[dstv.at[j]], add=True)

    pltpu.async_copy(idx_src(0), srcb.at[0], semIA)
    pltpu.async_copy(idx_src(1), srcb.at[1], semIB)
    plsc.subcore_barrier()
    pltpu.make_async_copy(idx_src(0), srcb.at[0], semIA).wait()
    fire(0, 0, rows0, asv0, adv0, semRA)

    def pair(t, carry):
        j0 = 2 * t
        j1 = j0 + 1
        pltpu.make_async_copy(idx_src(j1), srcb.at[1], semIB).wait()
        fire(j1, 1, rows1, asv1, adv1, semRB)
        drain(j0, 0, rows0, asv0, adv0, semRA)

        @pl.when(t < _NCH // 2 - 1)
        def _():
            pltpu.async_copy(idx_src(j0 + 2), srcb.at[0], semIA)

        compute(j0, rows0, asv0, adv0)

        @pl.when(t < _NCH // 2 - 1)
        def _():
            pltpu.make_async_copy(idx_src(j0 + 2), srcb.at[0], semIA).wait()
            fire(j0 + 2, 0, rows0, asv0, adv0, semRA)

        drain(j1, 1, rows1, asv1, adv1, semRB)

        @pl.when(t < _NCH // 2 - 1)
        def _():
            pltpu.async_copy(idx_src(j1 + 2), srcb.at[1], semIB)

        compute(j1, rows1, asv1, adv1)
        return carry

    lax.fori_loop(0, _NCH // 2, pair, 0)
    plsc.subcore_barrier()
    _drain_acc1d(accS, zbuf, s, c, outS_hbm)
    _drain_acc2d(accN, rows0, s, outN_hbm.at[c])


def _t1_body(x_ref, w_ref, degp_ref, hs_ref, dis_ref):
    h = jnp.dot(x_ref[...], w_ref[...], preferred_element_type=jnp.float32)
    deg = degp_ref[0] + degp_ref[1] + 1.0
    dis = lax.rsqrt(deg)
    dis_ref[...] = dis
    hs_ref[...] = dis[:, None] * h


def _t2_body(hs_ref, dis_ref, a0_ref, a1_ref, bg_ref, wgat_ref, vs_ref, vd_ref,
             h2_ref, asrc_ref, adst_ref, mb_ref, pself_ref):
    dis = dis_ref[...]
    agg = dis[:, None] * (a0_ref[...] + a1_ref[...] + hs_ref[...])
    h1 = jnp.maximum(agg + bg_ref[...][None, :], 0.0)
    h2 = jnp.dot(h1, wgat_ref[...], preferred_element_type=jnp.float32)
    h2_ref[...] = h2
    asrc = jnp.dot(h2, vs_ref[...][:, None], preferred_element_type=jnp.float32)[:, 0]
    adst = jnp.dot(h2, vd_ref[...][:, None], preferred_element_type=jnp.float32)[:, 0]
    asrc_ref[...] = asrc
    adst_ref[...] = adst
    mraw = jnp.max(asrc) + jnp.max(adst)
    m = jnp.where(mraw >= 0.0, mraw, 0.2 * mraw)
    mb_ref[...] = jnp.full((16,), m, jnp.float32)
    sself = asrc + adst
    sself = jnp.where(sself >= 0.0, sself, 0.2 * sself)
    pself_ref[...] = jnp.exp(sself - m)


def _t3_body(sp_ref, np_ref, h2_ref, pself_ref, bgat_ref,
             w1a_ref, b1a_ref, w2a_ref, b2a_ref,
             w1v_ref, b1v_ref, w2v_ref, b2v_ref, q_ref):
    pself = pself_ref[...]
    ssum = sp_ref[0] + sp_ref[1] + pself
    numer = np_ref[0] + np_ref[1] + pself[:, None] * h2_ref[...]
    h2 = jnp.maximum(numer / (ssum[:, None] + 1e-16) + bgat_ref[...][None, :], 0.0)
    g = jnp.sum(h2[:_N], axis=0, keepdims=True)
    ga = jnp.maximum(jnp.dot(g, w1a_ref[...], preferred_element_type=jnp.float32)
                     + b1a_ref[...][None, :], 0.0)
    a = jnp.dot(ga, w2a_ref[...], preferred_element_type=jnp.float32) + b2a_ref[...][None, :]
    gv = jnp.maximum(jnp.dot(g, w1v_ref[...], preferred_element_type=jnp.float32)
                     + b1v_ref[...][None, :], 0.0)
    v = jnp.dot(gv, w2v_ref[...], preferred_element_type=jnp.float32) + b2v_ref[...][None, :]
    q_ref[...] = v + a - jnp.mean(a, axis=1, keepdims=True)


def kernel(x, edge_index, edge_attr, W_gcn, b_gcn, W_gat, att_src, att_dst,
           b_gat, W1a, b1a, W2a, b2a, W1v, b1v, W2v, b2v):
    del edge_attr
    e = edge_index.shape[1]
    pad_e = _EPAD - e
    pad_idx = _N + (jnp.arange(pad_e, dtype=jnp.int32) % (_NPAD - _N))
    src = jnp.concatenate([edge_index[0], pad_idx])
    dst = jnp.concatenate([edge_index[1], pad_idx])
    dst_r = dst.reshape(_NW, _NCH, _CH)
    xp = jnp.concatenate([x, jnp.zeros((_NPAD - _N, _D), jnp.float32)], axis=0)

    degp = _deg_kernel(dst_r).reshape(2, _NPAD)

    hs, dis = pl.pallas_call(
        _t1_body,
        out_shape=(
            jax.ShapeDtypeStruct((_NPAD, _D), jnp.float32),
            jax.ShapeDtypeStruct((_NPAD,), jnp.float32),
        ),
    )(xp, W_gcn, degp)

    accg = _gcn_kernel(hs, src, dst_r)

    h2pre, asrc, adst, mb, pself = pl.pallas_call(
        _t2_body,
        out_shape=(
            jax.ShapeDtypeStruct((_NPAD, _D), jnp.float32),
            jax.ShapeDtypeStruct((_NPAD,), jnp.float32),
            jax.ShapeDtypeStruct((_NPAD,), jnp.float32),
            jax.ShapeDtypeStruct((16,), jnp.float32),
            jax.ShapeDtypeStruct((_NPAD,), jnp.float32),
        ),
    )(hs, dis, accg[0], accg[1], b_gcn, W_gat, att_src, att_dst)

    ssump, numerp = _gat_kernel(h2pre, asrc, adst, mb, src, dst_r)
    ssump = ssump.reshape(2, _NPAD)

    q = pl.pallas_call(
        _t3_body,
        out_shape=jax.ShapeDtypeStruct((1, 5), jnp.float32),
    )(ssump, numerp, h2pre, pself, b_gat,
      W1a, b1a, W2a, b2a, W1v, b1v, W2v, b2v)
    return q


# speedup vs baseline: 56.9677x; 1.0213x over previous
import functools

import jax
import jax.numpy as jnp
from jax import lax
from jax.experimental import pallas as pl
from jax.experimental.pallas import tpu as pltpu
from jax.experimental.pallas import tpu_sc as plsc

_N = 10000
_D = 128
_NPAD = 10112
_NW = 32
_CH = 128
_NCH = 80
_EPW = _NCH * _CH
_EPAD = _NW * _EPW
_ZS = _NPAD // 16

_SLABS = tuple((off, min(_CH, _ZS - off)) for off in range(0, _ZS, _CH))


def _zero_rows(rows, nrows):
    z16 = jnp.zeros((16,), jnp.float32)

    def z(r, carry):
        for k in range(_D // 16):
            rows[r, pl.ds(k * 16, 16)] = z16
        return carry

    lax.fori_loop(0, nrows, z, 0)


def _zero_acc2d(acc, rows, s):
    _zero_rows(rows, _CH)
    for off, sz in _SLABS:
        pltpu.sync_copy(rows.at[pl.ds(0, sz)], acc.at[pl.ds(s * _ZS + off, sz)])


def _drain_acc2d(acc, rows, s, out_ref):
    for off, sz in _SLABS:
        pltpu.sync_copy(acc.at[pl.ds(s * _ZS + off, sz)], rows.at[pl.ds(0, sz)])
        pltpu.sync_copy(rows.at[pl.ds(0, sz)], out_ref.at[pl.ds(s * _ZS + off, sz)])


def _zero_acc1d(acc, buf, s):
    z16 = jnp.zeros((16,), jnp.float32)

    def z(r, carry):
        buf[pl.ds(r * 16, 16)] = z16
        return carry

    lax.fori_loop(0, 640 // 16, z, 0)
    pltpu.sync_copy(buf.at[pl.ds(0, _ZS)], acc.at[pl.ds(s * _ZS, _ZS)])


def _drain_acc1d(acc, buf, s, c, out_hbm):
    pltpu.sync_copy(acc.at[pl.ds(s * _ZS, _ZS)], buf.at[pl.ds(0, _ZS)])
    pltpu.sync_copy(buf.at[pl.ds(0, _ZS)],
                    out_hbm.at[pl.ds(c * _NPAD + s * _ZS, _ZS)])


def _scale_rows(rows, wv):
    def group(i, carry):
        base = i * 16
        w16 = wv[pl.ds(base, 16)]
        for e in range(16):
            w = w16[e]
            for r in range(_D // 16):
                sl = pl.ds(r * 16, 16)
                rows[base + e, sl] = rows[base + e, sl] * w
        return carry

    lax.fori_loop(0, _CH // 16, group, 0)


_mesh = plsc.VectorSubcoreMesh(
    core_axis_name="c", subcore_axis_name="s", num_cores=2, num_subcores=16
)


@functools.partial(
    pl.kernel,
    out_type=jax.ShapeDtypeStruct((2 * _NPAD,), jnp.float32),
    mesh=_mesh,
    scratch_types=[
        pltpu.VMEM((_NCH, _CH), jnp.int32),
        pltpu.VMEM((_CH,), jnp.float32),
        pltpu.VMEM((640,), jnp.float32),
        pltpu.VMEM_SHARED((_NPAD,), jnp.float32),
    ],
)
def _deg_kernel(dst_hbm, out_hbm, dstv, onesv, zbuf, acc):
    c = lax.axis_index("c")
    s = lax.axis_index("s")
    wid = c * 16 + s
    _zero_acc1d(acc, zbuf, s)
    for i in range(_CH // 16):
        onesv[pl.ds(i * 16, 16)] = jnp.full((16,), 1.0, jnp.float32)
    pltpu.sync_copy(dst_hbm.at[wid], dstv)
    plsc.subcore_barrier()

    def chunk(j, carry):
        pltpu.sync_copy(onesv, acc.at[dstv.at[j]], add=True)
        return carry

    lax.fori_loop(0, _NCH, chunk, 0)
    plsc.subcore_barrier()
    _drain_acc1d(acc, zbuf, s, c, out_hbm)


@functools.partial(
    pl.kernel,
    out_type=jax.ShapeDtypeStruct((2, _NPAD, _D), jnp.float32),
    mesh=_mesh,
    scratch_types=[
        pltpu.VMEM((_NCH, _CH), jnp.int32),
        pltpu.VMEM((2, _CH), jnp.int32),
        pltpu.VMEM((_CH, _D), jnp.float32),
        pltpu.VMEM((_CH, _D), jnp.float32),
        pltpu.SemaphoreType.DMA,
        pltpu.SemaphoreType.DMA,
        pltpu.SemaphoreType.DMA,
        pltpu.SemaphoreType.DMA,
        pltpu.VMEM_SHARED((_NPAD, _D), jnp.float32),
    ],
)
def _gcn_kernel(hs_hbm, srcf_hbm, dst_hbm, out_hbm,
                dstv, srcb, rows0, rows1, semIA, semIB, semRA, semRB, acc):
    c = lax.axis_index("c")
    s = lax.axis_index("s")
    wid = c * 16 + s
    _zero_acc2d(acc, rows0, s)
    pltpu.sync_copy(dst_hbm.at[wid], dstv)
    base = wid * _NCH * _CH

    def idx_src(j):
        return srcf_hbm.at[pl.ds(base + j * _CH, _CH)]

    pltpu.async_copy(idx_src(0), srcb.at[0], semIA)
    pltpu.async_copy(idx_src(1), srcb.at[1], semIB)
    plsc.subcore_barrier()
    pltpu.make_async_copy(idx_src(0), srcb.at[0], semIA).wait()
    pltpu.async_copy(hs_hbm.at[srcb.at[0]], rows0, semRA)

    def pair(t, carry):
        j0 = 2 * t
        j1 = j0 + 1
        pltpu.make_async_copy(idx_src(j1), srcb.at[1], semIB).wait()
        pltpu.async_copy(hs_hbm.at[srcb.at[1]], rows1, semRB)
        pltpu.make_async_copy(hs_hbm.at[srcb.at[0]], rows0, semRA).wait()

        @pl.when(t < _NCH // 2 - 1)
        def _():
            pltpu.async_copy(idx_src(j0 + 2), srcb.at[0], semIA)

        pltpu.sync_copy(rows0, acc.at[dstv.at[j0]], add=True)

        @pl.when(t < _NCH // 2 - 1)
        def _():
            pltpu.make_async_copy(idx_src(j0 + 2), srcb.at[0], semIA).wait()
            pltpu.async_copy(hs_hbm.at[srcb.at[0]], rows0, semRA)

        pltpu.make_async_copy(hs_hbm.at[srcb.at[1]], rows1, semRB).wait()

        @pl.when(t < _NCH // 2 - 1)
        def _():
            pltpu.async_copy(idx_src(j1 + 2), srcb.at[1], semIB)

        pltpu.sync_copy(rows1, acc.at[dstv.at[j1]], add=True)
        return carry

    lax.fori_loop(0, _NCH // 2, pair, 0)
    plsc.subcore_barrier()
    _drain_acc2d(acc, rows0, s, out_hbm.at[c])


@functools.partial(
    pl.kernel,
    out_type=(
        jax.ShapeDtypeStruct((2 * _NPAD,), jnp.float32),
        jax.ShapeDtypeStruct((2, _NPAD, _D), jnp.float32),
    ),
    mesh=_mesh,
    scratch_types=[
        pltpu.VMEM((_NCH, _CH), jnp.int32),
        pltpu.VMEM((2, _CH), jnp.int32),
        pltpu.VMEM((_CH,), jnp.float32),
        pltpu.VMEM((_CH,), jnp.float32),
        pltpu.VMEM((_CH,), jnp.float32),
        pltpu.VMEM((_CH,), jnp.float32),
        pltpu.VMEM((_CH,), jnp.float32),
        pltpu.VMEM((_CH, _D), jnp.float32),
        pltpu.VMEM((_CH, _D), jnp.float32),
        pltpu.VMEM((16,), jnp.float32),
        pltpu.VMEM((640,), jnp.float32),
        pltpu.SemaphoreType.DMA,
        pltpu.SemaphoreType.DMA,
        pltpu.SemaphoreType.DMA,
        pltpu.SemaphoreType.DMA,
        pltpu.VMEM_SHARED((_NPAD,), jnp.float32),
        pltpu.VMEM_SHARED((_NPAD, _D), jnp.float32),
    ],
)
def _gat_kernel(h2_hbm, asrc_hbm, adst_hbm, mb_hbm, srcf_hbm, dst_hbm,
                outS_hbm, outN_hbm,
                dstv, srcb, asv0, adv0, asv1, adv1, pv, rows0, rows1, mv,
                zbuf, semIA, semIB, semRA, semRB, accS, accN):
    c = lax.axis_index("c")
    s = lax.axis_index("s")
    wid = c * 16 + s
    _zero_acc1d(accS, zbuf, s)
    _zero_acc2d(accN, rows0, s)
    pltpu.sync_copy(mb_hbm, mv)
    pltpu.sync_copy(dst_hbm.at[wid], dstv)
    base = wid * _NCH * _CH

    def idx_src(j):
        return srcf_hbm.at[pl.ds(base + j * _CH, _CH)]

    def fire(j, b, rows_b, asv_b, adv_b, sem):
        pltpu.async_copy(h2_hbm.at[srcb.at[b]], rows_b, sem)
        pltpu.async_copy(asrc_hbm.at[srcb.at[b]], asv_b, sem)
        pltpu.async_copy(adst_hbm.at[dstv.at[j]], adv_b, sem)

    def drain(j, b, rows_b, asv_b, adv_b, sem):
        pltpu.make_async_copy(h2_hbm.at[srcb.at[b]], rows_b, sem).wait()
        pltpu.make_async_copy(asrc_hbm.at[srcb.at[b]], asv_b, sem).wait()
        pltpu.make_async_copy(adst_hbm.at[dstv.at[j]], adv_b, sem).wait()

    def compute(j, rows_b, asv_b, adv_b):
        m16 = mv[...]
        for i in range(_CH // 16):
            sl = pl.ds(i * 16, 16)
            sc = asv_b[sl] + adv_b[sl]
            sc = jnp.where(sc >= 0.0, sc, 0.2 * sc)
            pv[sl] = jnp.exp(sc - m16)
        _scale_rows(rows_b, pv)
        pltpu.sync_copy(pv, accS.at[dstv.at[j]], add=True)
        pltpu.sync_copy(rows_b, accN.at[dstv.at[j]], add=True)

    pltpu.async_copy(idx_src(0), srcb.at[0], semIA)
    pltpu.async_copy(idx_src(1), srcb.at[1], semIB)
    plsc.subcore_barrier()
    pltpu.make_async_copy(idx_src(0), srcb.at[0], semIA).wait()
    fire(0, 0, rows0, asv0, adv0, semRA)

    def pair(t, carry):
        j0 = 2 * t
        j1 = j0 + 1
        pltpu.make_async_copy(idx_src(j1), srcb.at[1], semIB).wait()
        fire(j1, 1, rows1, asv1, adv1, semRB)
        drain(j0, 0, rows0, asv0, adv0, semRA)

        @pl.when(t < _NCH // 2 - 1)
        def _():
            pltpu.async_copy(idx_src(j0 + 2), srcb.at[0], semIA)

        compute(j0, rows0, asv0, adv0)

        @pl.when(t < _NCH // 2 - 1)
        def _():
            pltpu.make_async_copy(idx_src(j0 + 2), srcb.at[0], semIA).wait()
            fire(j0 + 2, 0, rows0, asv0, adv0, semRA)

        drain(j1, 1, rows1, asv1, adv1, semRB)

        @pl.when(t < _NCH // 2 - 1)
        def _():
            pltpu.async_copy(idx_src(j1 + 2), srcb.at[1], semIB)

        compute(j1, rows1, asv1, adv1)
        return carry

    lax.fori_loop(0, _NCH // 2, pair, 0)
    plsc.subcore_barrier()
    _drain_acc1d(accS, zbuf, s, c, outS_hbm)
    _drain_acc2d(accN, rows0, s, outN_hbm.at[c])


def _t1_body(x_ref, w_ref, degp_ref, hs_ref, dis_ref):
    h = jnp.dot(x_ref[...], w_ref[...], preferred_element_type=jnp.float32)
    deg = degp_ref[0] + degp_ref[1] + 1.0
    dis = lax.rsqrt(deg)
    dis_ref[...] = dis
    hs_ref[...] = dis[:, None] * h


def _t2_body(hs_ref, dis_ref, accp_ref, bg_ref, wgat_ref, vs_ref, vd_ref,
             h2_ref, asrc_ref, adst_ref, mb_ref, pself_ref):
    dis = dis_ref[...]
    agg = dis[:, None] * (accp_ref[0] + accp_ref[1] + hs_ref[...])
    h1 = jnp.maximum(agg + bg_ref[...][None, :], 0.0)
    h2 = jnp.dot(h1, wgat_ref[...], preferred_element_type=jnp.float32)
    h2_ref[...] = h2
    asrc = jnp.dot(h2, vs_ref[...][:, None], preferred_element_type=jnp.float32)[:, 0]
    adst = jnp.dot(h2, vd_ref[...][:, None], preferred_element_type=jnp.float32)[:, 0]
    asrc_ref[...] = asrc
    adst_ref[...] = adst
    mraw = jnp.max(asrc) + jnp.max(adst)
    m = jnp.where(mraw >= 0.0, mraw, 0.2 * mraw)
    mb_ref[...] = jnp.full((16,), m, jnp.float32)
    sself = asrc + adst
    sself = jnp.where(sself >= 0.0, sself, 0.2 * sself)
    pself_ref[...] = jnp.exp(sself - m)


def _t3_body(sp_ref, np_ref, h2_ref, pself_ref, bgat_ref,
             w1a_ref, b1a_ref, w2a_ref, b2a_ref,
             w1v_ref, b1v_ref, w2v_ref, b2v_ref, q_ref):
    pself = pself_ref[...]
    sp = sp_ref[...]
    ssum = sp[:_NPAD] + sp[_NPAD:] + pself
    numer = np_ref[0] + np_ref[1] + pself[:, None] * h2_ref[...]
    h2 = jnp.maximum(numer / (ssum[:, None] + 1e-16) + bgat_ref[...][None, :], 0.0)
    g = jnp.sum(h2[:_N], axis=0, keepdims=True)
    ga = jnp.maximum(jnp.dot(g, w1a_ref[...], preferred_element_type=jnp.float32)
                     + b1a_ref[...][None, :], 0.0)
    a = jnp.dot(ga, w2a_ref[...], preferred_element_type=jnp.float32) + b2a_ref[...][None, :]
    gv = jnp.maximum(jnp.dot(g, w1v_ref[...], preferred_element_type=jnp.float32)
                     + b1v_ref[...][None, :], 0.0)
    v = jnp.dot(gv, w2v_ref[...], preferred_element_type=jnp.float32) + b2v_ref[...][None, :]
    q_ref[...] = v + a - jnp.mean(a, axis=1, keepdims=True)


def kernel(x, edge_index, edge_attr, W_gcn, b_gcn, W_gat, att_src, att_dst,
           b_gat, W1a, b1a, W2a, b2a, W1v, b1v, W2v, b2v):
    del edge_attr
    e = edge_index.shape[1]
    pad_e = _EPAD - e
    pad_idx = _N + (jnp.arange(pad_e, dtype=jnp.int32) % (_NPAD - _N))
    src = jnp.concatenate([edge_index[0], pad_idx])
    dst = jnp.concatenate([edge_index[1], pad_idx])
    dst_r = dst.reshape(_NW, _NCH, _CH)
    xp = jnp.concatenate([x, jnp.zeros((_NPAD - _N, _D), jnp.float32)], axis=0)

    degp = _deg_kernel(dst_r).reshape(2, _NPAD)

    hs, dis = pl.pallas_call(
        _t1_body,
        out_shape=(
            jax.ShapeDtypeStruct((_NPAD, _D), jnp.float32),
            jax.ShapeDtypeStruct((_NPAD,), jnp.float32),
        ),
    )(xp, W_gcn, degp)

    accg = _gcn_kernel(hs, src, dst_r)

    h2pre, asrc, adst, mb, pself = pl.pallas_call(
        _t2_body,
        out_shape=(
            jax.ShapeDtypeStruct((_NPAD, _D), jnp.float32),
            jax.ShapeDtypeStruct((_NPAD,), jnp.float32),
            jax.ShapeDtypeStruct((_NPAD,), jnp.float32),
            jax.ShapeDtypeStruct((16,), jnp.float32),
            jax.ShapeDtypeStruct((_NPAD,), jnp.float32),
        ),
    )(hs, dis, accg, b_gcn, W_gat, att_src, att_dst)

    ssump, numerp = _gat_kernel(h2pre, asrc, adst, mb, src, dst_r)

    q = pl.pallas_call(
        _t3_body,
        out_shape=jax.ShapeDtypeStruct((1, 5), jnp.float32),
    )(ssump, numerp, h2pre, pself, b_gat,
      W1a, b1a, W2a, b2a, W1v, b1v, W2v, b2v)
    return q
